# baseline (speedup 1.0000x reference)
import functools

import jax
import jax.numpy as jnp
from jax import lax
from jax.experimental import pallas as pl
from jax.experimental.pallas import tpu as pltpu
from jax.experimental.pallas import tpu_sc as plsc

NUM_CLASSES = 100000
FEAT = 64
BATCH = 16384
NUM_CORES = 2
NUM_SUBCORES = 16
LANES = 16
NW = NUM_CORES * NUM_SUBCORES
FPW = FEAT // NW
CHUNK = 2048
NCHUNKS = BATCH // CHUNK
GROUPS = CHUNK // LANES


def _sc_partials(xt, labels, cent):
    mesh = plsc.VectorSubcoreMesh(core_axis_name="c", subcore_axis_name="s")

    @functools.partial(
        pl.kernel,
        mesh=mesh,
        out_type=jax.ShapeDtypeStruct((FEAT, BATCH), jnp.float32),
        compiler_params=pltpu.CompilerParams(
            needs_layout_passes=False, use_tc_tiling_on_sc=True
        ),
        scratch_types=[
            pltpu.VMEM((NUM_CLASSES,), jnp.float32),
            pltpu.VMEM((2, CHUNK), jnp.int32),
            pltpu.VMEM((2, CHUNK), jnp.float32),
            pltpu.VMEM((2, CHUNK), jnp.float32),
            pltpu.SemaphoreType.DMA,
            pltpu.SemaphoreType.DMA,
            pltpu.SemaphoreType.DMA,
            pltpu.SemaphoreType.DMA,
        ],
    )
    def k(xt_hbm, lab_hbm, cen_hbm, out_hbm, tab_v, lab_v, x_v, o_v,
          sem_t, sem0, sem1, sem_o):
        wid = lax.axis_index("s") * NUM_CORES + lax.axis_index("c")
        sems = (sem0, sem1)

        for p in range(FPW):
            f = wid + p * NW
            tab_cp = pltpu.async_copy(cen_hbm.at[f], tab_v, sem_t)
            pending = [
                pltpu.async_copy(
                    lab_hbm.at[pl.ds(0, CHUNK)], lab_v.at[0], sems[0]
                ),
                pltpu.async_copy(
                    xt_hbm.at[f, pl.ds(0, CHUNK)], x_v.at[0], sems[0]
                ),
            ]
            out_pending = [None, None]
            tab_cp.wait()
            for ch in range(NCHUNKS):
                buf = ch % 2
                nbuf = (ch + 1) % 2
                if ch + 1 < NCHUNKS:
                    nxt = [
                        pltpu.async_copy(
                            lab_hbm.at[pl.ds((ch + 1) * CHUNK, CHUNK)],
                            lab_v.at[nbuf], sems[nbuf],
                        ),
                        pltpu.async_copy(
                            xt_hbm.at[f, pl.ds((ch + 1) * CHUNK, CHUNK)],
                            x_v.at[nbuf], sems[nbuf],
                        ),
                    ]
                else:
                    nxt = []
                for cp in pending:
                    cp.wait()
                pending = nxt
                if out_pending[buf] is not None:
                    out_pending[buf].wait()

                @plsc.parallel_loop(0, GROUPS, unroll=8)
                def group_body(g, buf=buf):
                    off = g * LANES
                    idx = lab_v[buf, pl.ds(off, LANES)]
                    cg = plsc.load_gather(tab_v, [idx])
                    xv = x_v[buf, pl.ds(off, LANES)]
                    d = xv - cg
                    o_v[buf, pl.ds(off, LANES)] = d * d

                out_pending[buf] = pltpu.async_copy(
                    o_v.at[buf], out_hbm.at[f, pl.ds(ch * CHUNK, CHUNK)], sem_o
                )
            for cp in out_pending:
                if cp is not None:
                    cp.wait()

    return k(xt, labels, cent)


def _tc_reduce(partials):
    def body(p_ref, o_ref):
        dist = jnp.sum(p_ref[...], axis=0)
        dist = jnp.minimum(jnp.maximum(dist, 1e-12), 1e12)
        o_ref[0, 0] = jnp.sum(dist) * (1.0 / BATCH)

    return pl.pallas_call(
        body,
        out_shape=jax.ShapeDtypeStruct((1, 1), jnp.float32),
        out_specs=pl.BlockSpec(memory_space=pltpu.SMEM),
    )(partials)


def kernel(x, labels, centers):
    partials = _sc_partials(x.T, labels.astype(jnp.int32), centers.T)
    return _tc_reduce(partials)[0, 0]


# speedup vs baseline: 1.0633x; 1.0633x over previous
import functools

import jax
import jax.numpy as jnp
from jax import lax
from jax.experimental import pallas as pl
from jax.experimental.pallas import tpu as pltpu
from jax.experimental.pallas import tpu_sc as plsc

NUM_CLASSES = 100000
FEAT = 64
BATCH = 16384
NUM_CORES = 2
NUM_SUBCORES = 16
LANES = 16
NW = NUM_CORES * NUM_SUBCORES
FPW = FEAT // NW
CHUNK = 2048
NCHUNKS = BATCH // CHUNK
GROUPS = CHUNK // LANES


def _sc_partials(xt, labels, cent):
    mesh = plsc.VectorSubcoreMesh(core_axis_name="c", subcore_axis_name="s")

    @functools.partial(
        pl.kernel,
        mesh=mesh,
        out_type=jax.ShapeDtypeStruct((NW, BATCH), jnp.float32),
        compiler_params=pltpu.CompilerParams(
            needs_layout_passes=False, use_tc_tiling_on_sc=True
        ),
        scratch_types=[
            pltpu.VMEM((NUM_CLASSES,), jnp.float32),
            pltpu.VMEM((2, CHUNK), jnp.int32),
            pltpu.VMEM((2, CHUNK), jnp.float32),
            pltpu.VMEM((BATCH,), jnp.float32),
            pltpu.SemaphoreType.DMA,
            pltpu.SemaphoreType.DMA,
            pltpu.SemaphoreType.DMA,
        ],
    )
    def k(xt_hbm, lab_hbm, cen_hbm, out_hbm, tab_v, lab_v, x_v, acc_v,
          sem_t, sem0, sem1):
        wid = lax.axis_index("s") * NUM_CORES + lax.axis_index("c")
        sems = (sem0, sem1)

        for p in range(FPW):
            f = wid + p * NW
            tab_cp = pltpu.async_copy(cen_hbm.at[f], tab_v, sem_t)
            pending = [
                pltpu.async_copy(
                    lab_hbm.at[pl.ds(0, CHUNK)], lab_v.at[0], sems[0]
                ),
                pltpu.async_copy(
                    xt_hbm.at[f, pl.ds(0, CHUNK)], x_v.at[0], sems[0]
                ),
            ]
            tab_cp.wait()
            for ch in range(NCHUNKS):
                buf = ch % 2
                nbuf = (ch + 1) % 2
                if ch + 1 < NCHUNKS:
                    nxt = [
                        pltpu.async_copy(
                            lab_hbm.at[pl.ds((ch + 1) * CHUNK, CHUNK)],
                            lab_v.at[nbuf], sems[nbuf],
                        ),
                        pltpu.async_copy(
                            xt_hbm.at[f, pl.ds((ch + 1) * CHUNK, CHUNK)],
                            x_v.at[nbuf], sems[nbuf],
                        ),
                    ]
                else:
                    nxt = []
                for cp in pending:
                    cp.wait()
                pending = nxt

                @plsc.parallel_loop(0, GROUPS, unroll=8)
                def group_body(g, ch=ch, p=p, buf=buf):
                    off = g * LANES
                    idx = lab_v[buf, pl.ds(off, LANES)]
                    cg = plsc.load_gather(tab_v, [idx])
                    xv = x_v[buf, pl.ds(off, LANES)]
                    d = xv - cg
                    d2 = d * d
                    aoff = ch * CHUNK + off
                    if p == 0:
                        acc_v[pl.ds(aoff, LANES)] = d2
                    else:
                        acc_v[pl.ds(aoff, LANES)] = acc_v[pl.ds(aoff, LANES)] + d2
        pltpu.sync_copy(acc_v, out_hbm.at[wid])

    return k(xt, labels, cent)


def _tc_reduce(partials):
    def body(p_ref, o_ref):
        dist = jnp.sum(p_ref[...], axis=0)
        dist = jnp.minimum(jnp.maximum(dist, 1e-12), 1e12)
        o_ref[0, 0] = jnp.sum(dist) * (1.0 / BATCH)

    return pl.pallas_call(
        body,
        out_shape=jax.ShapeDtypeStruct((1, 1), jnp.float32),
        out_specs=pl.BlockSpec(memory_space=pltpu.SMEM),
    )(partials)


def kernel(x, labels, centers):
    partials = _sc_partials(x.T, labels.astype(jnp.int32), centers.T)
    return _tc_reduce(partials)[0, 0]
